# baseline (speedup 1.0000x reference)
import jax
import jax.numpy as jnp
from jax import lax
from jax.experimental import pallas as pl
from jax.experimental.pallas import tpu as pltpu
from jax.experimental.pallas import tpu_sc as plsc

_B = 16384
_NC = 2
_NS = 16
_NW = _NC * _NS
_BPW = _B // _NW
_NG = _BPW // 16
_RC = 4096
_NSTEPC = _B // _RC
_FS = 129


def _gather_body(t_hbm, m_hbm, d_hbm, w_hbm, h_hbm, out_hbm,
                 t_v, mi_v, di_v, wi_v, hi_v, cb_v):
    wid = lax.axis_index("s") * _NC + lax.axis_index("c")
    base = wid * _BPW
    pltpu.sync_copy(t_hbm, t_v)
    pltpu.sync_copy(m_hbm.at[pl.ds(base, _BPW)], mi_v)
    pltpu.sync_copy(d_hbm.at[pl.ds(base, _BPW)], di_v)
    pltpu.sync_copy(w_hbm.at[pl.ds(base, _BPW)], wi_v)
    pltpu.sync_copy(h_hbm.at[pl.ds(base, _BPW)], hi_v)

    @plsc.parallel_loop(0, _NG, unroll=2)
    def g_body(g):
        o = g * 16
        mi = mi_v[pl.ds(o, 16)] * _FS
        di = (di_v[pl.ds(o, 16)] + 13) * _FS
        wi = (wi_v[pl.ds(o, 16)] + 45) * _FS
        hi = (hi_v[pl.ds(o, 16)] + 52) * _FS
        for k in range(128):
            e = ((plsc.load_gather(t_v, [mi + k])
                  + plsc.load_gather(t_v, [di + k]))
                 + plsc.load_gather(t_v, [wi + k])
                 + plsc.load_gather(t_v, [hi + k]))
            cb_v[k, pl.ds(o, 16)] = e

    pltpu.sync_copy(cb_v, out_hbm.at[:, pl.ds(base, _BPW)])


def _mlp_body(cb_ref, w1_ref, b1_ref, w2_ref, b2_ref, w3_ref, b3_ref,
              out_ref):
    f32 = jnp.float32
    c00 = (((0,), (0,)), ((), ()))
    h1 = jax.lax.dot_general(w1_ref[...], cb_ref[...], c00,
                             preferred_element_type=f32)
    h1 = jnp.maximum(h1 + b1_ref[...], 0.0)
    h2 = jax.lax.dot_general(w2_ref[...], h1, c00,
                             preferred_element_type=f32)
    h2 = jnp.maximum(h2 + b2_ref[...], 0.0)
    o = jax.lax.dot_general(w3_ref[...], h2, c00,
                            preferred_element_type=f32)
    out_ref[...] = jnp.maximum(o + b3_ref[...], 0.0)


def kernel(month, day, weekday, hour, month_table, day_table, weekday_table,
           hour_table, W1, b1, W2, b2, W3, b3):
    i32 = jnp.int32
    f32 = jnp.float32
    m = month.astype(i32)
    d = day.astype(i32)
    w = weekday.astype(i32)
    h = hour.astype(i32)
    tcat = jnp.concatenate(
        [month_table, day_table, weekday_table, hour_table,
         jnp.zeros((52, 128), f32)], axis=0)
    tflat = jnp.pad(tcat, ((0, 0), (0, _FS - 128))).reshape(128 * _FS)

    mesh = plsc.VectorSubcoreMesh(core_axis_name="c", subcore_axis_name="s")
    comb = pl.kernel(
        _gather_body,
        out_type=jax.ShapeDtypeStruct((128, _B), f32),
        mesh=mesh,
        compiler_params=pltpu.CompilerParams(needs_layout_passes=False),
        scratch_types=[
            pltpu.VMEM((128 * _FS,), f32),
            pltpu.VMEM((_BPW,), i32),
            pltpu.VMEM((_BPW,), i32),
            pltpu.VMEM((_BPW,), i32),
            pltpu.VMEM((_BPW,), i32),
            pltpu.VMEM((128, _BPW), f32),
        ],
    )(tflat, m, d, w, h)

    b1c = b1.reshape(64, 1)
    b2c = b2.reshape(32, 1)
    b3c = b3.reshape(1, 1)
    full = lambda s: pl.BlockSpec(s, lambda i: tuple(0 for _ in s))
    out = pl.pallas_call(
        _mlp_body,
        grid=(_NSTEPC,),
        in_specs=[pl.BlockSpec((128, _RC), lambda i: (0, i)),
                  full((128, 64)), full((64, 1)), full((64, 32)),
                  full((32, 1)), full((32, 1)), full((1, 1))],
        out_specs=pl.BlockSpec((1, _RC), lambda i: (0, i)),
        out_shape=jax.ShapeDtypeStruct((1, _B), f32),
    )(comb, W1, b1c, W2, b2c, W3, b3c)
    return out.reshape(_B, 1)


# speedup vs baseline: 2.0332x; 2.0332x over previous
import jax
import jax.numpy as jnp
from jax import lax
from jax.experimental import pallas as pl
from jax.experimental.pallas import tpu as pltpu
from jax.experimental.pallas import tpu_sc as plsc

_B = 16384
_BSC = 4096
_BTC = _B - _BSC
_NC = 2
_NS = 16
_NW = _NC * _NS
_BPW = _BSC // _NW
_NG = _BPW // 16
_FS = 129
_R = 4096
_NSTEP_TC = _BTC // _R


def _gather_body(t_hbm, m_hbm, d_hbm, w_hbm, h_hbm, out_hbm,
                 t_v, mi_v, di_v, wi_v, hi_v, cb_v):
    wid = lax.axis_index("s") * _NC + lax.axis_index("c")
    base = wid * _BPW
    pltpu.sync_copy(t_hbm, t_v)
    pltpu.sync_copy(m_hbm.at[pl.ds(base, _BPW)], mi_v)
    pltpu.sync_copy(d_hbm.at[pl.ds(base, _BPW)], di_v)
    pltpu.sync_copy(w_hbm.at[pl.ds(base, _BPW)], wi_v)
    pltpu.sync_copy(h_hbm.at[pl.ds(base, _BPW)], hi_v)

    @plsc.parallel_loop(0, _NG)
    def g_body(g):
        o = g * 16
        mi = mi_v[pl.ds(o, 16)] * _FS
        di = (di_v[pl.ds(o, 16)] + 13) * _FS
        wi = (wi_v[pl.ds(o, 16)] + 45) * _FS
        hi = (hi_v[pl.ds(o, 16)] + 52) * _FS
        for k in range(128):
            e = ((plsc.load_gather(t_v, [mi + k])
                  + plsc.load_gather(t_v, [di + k]))
                 + plsc.load_gather(t_v, [wi + k])
                 + plsc.load_gather(t_v, [hi + k]))
            cb_v[k, pl.ds(o, 16)] = e

    pltpu.sync_copy(cb_v, out_hbm.at[:, pl.ds(base, _BPW)])


def _mlp(comb, w1_ref, b1_ref, w2_ref, b2_ref, w3_ref, b3_ref):
    f32 = jnp.float32
    c00 = (((0,), (0,)), ((), ()))
    h1 = jax.lax.dot_general(w1_ref[...], comb, c00,
                             preferred_element_type=f32)
    h1 = jnp.maximum(h1 + b1_ref[...], 0.0)
    h2 = jax.lax.dot_general(w2_ref[...], h1, c00,
                             preferred_element_type=f32)
    h2 = jnp.maximum(h2 + b2_ref[...], 0.0)
    o = jax.lax.dot_general(w3_ref[...], h2, c00,
                            preferred_element_type=f32)
    return jnp.maximum(o + b3_ref[...], 0.0)


def _mlp_body(cb_ref, w1_ref, b1_ref, w2_ref, b2_ref, w3_ref, b3_ref,
              out_ref):
    out_ref[...] = _mlp(cb_ref[...], w1_ref, b1_ref, w2_ref, b2_ref,
                        w3_ref, b3_ref)


def _hot_body(m_ref, d_ref, w_ref, h_ref, tcat_ref, w1_ref, b1_ref, w2_ref,
              b2_ref, w3_ref, b3_ref, out_ref):
    f32 = jnp.float32
    m = m_ref[0]
    d = d_ref[0]
    w = w_ref[0]
    h = h_ref[0]
    iota = jax.lax.broadcasted_iota(jnp.int32, (128, _R), 0)
    hot = ((iota == m) | (iota == d + 13) | (iota == w + 45)
           | (iota == h + 52))
    mh = jnp.where(hot, f32(1.0), f32(0.0))

    c00 = (((0,), (0,)), ((), ()))
    tcat = tcat_ref[...]
    t_hi = tcat.astype(jnp.bfloat16).astype(f32)
    t_lo = tcat - t_hi
    comb = (jax.lax.dot_general(t_hi, mh, c00, preferred_element_type=f32)
            + jax.lax.dot_general(t_lo, mh, c00,
                                  preferred_element_type=f32))
    o = _mlp(comb, w1_ref, b1_ref, w2_ref, b2_ref, w3_ref, b3_ref)
    out_ref[...] = o.reshape(1, 1, _R)


def kernel(month, day, weekday, hour, month_table, day_table, weekday_table,
           hour_table, W1, b1, W2, b2, W3, b3):
    i32 = jnp.int32
    f32 = jnp.float32
    m = month.astype(i32)
    d = day.astype(i32)
    w = weekday.astype(i32)
    h = hour.astype(i32)
    tcat = jnp.concatenate(
        [month_table, day_table, weekday_table, hour_table,
         jnp.zeros((52, 128), f32)], axis=0)
    tflat = jnp.pad(tcat, ((0, 0), (0, _FS - 128))).reshape(128 * _FS)
    b1c = b1.reshape(64, 1)
    b2c = b2.reshape(32, 1)
    b3c = b3.reshape(1, 1)

    mesh = plsc.VectorSubcoreMesh(core_axis_name="c", subcore_axis_name="s")
    comb_sc = pl.kernel(
        _gather_body,
        out_type=jax.ShapeDtypeStruct((128, _BSC), f32),
        mesh=mesh,
        compiler_params=pltpu.CompilerParams(needs_layout_passes=False),
        scratch_types=[
            pltpu.VMEM((128 * _FS,), f32),
            pltpu.VMEM((_BPW,), i32),
            pltpu.VMEM((_BPW,), i32),
            pltpu.VMEM((_BPW,), i32),
            pltpu.VMEM((_BPW,), i32),
            pltpu.VMEM((128, _BPW), f32),
        ],
    )(tflat, m[:_BSC], d[:_BSC], w[:_BSC], h[:_BSC])

    full = lambda s: pl.BlockSpec(s, lambda i: tuple(0 for _ in s))

    mt = m[_BSC:].reshape(_NSTEP_TC, 1, _R)
    dt = d[_BSC:].reshape(_NSTEP_TC, 1, _R)
    wt = w[_BSC:].reshape(_NSTEP_TC, 1, _R)
    ht = h[_BSC:].reshape(_NSTEP_TC, 1, _R)
    idx_spec = pl.BlockSpec((1, 1, _R), lambda i: (i, 0, 0))
    out_tc = pl.pallas_call(
        _hot_body,
        grid=(_NSTEP_TC,),
        in_specs=[idx_spec, idx_spec, idx_spec, idx_spec,
                  full((128, 128)), full((128, 64)), full((64, 1)),
                  full((64, 32)), full((32, 1)), full((32, 1)),
                  full((1, 1))],
        out_specs=pl.BlockSpec((1, 1, _R), lambda i: (i, 0, 0)),
        out_shape=jax.ShapeDtypeStruct((_NSTEP_TC, 1, _R), f32),
    )(mt, dt, wt, ht, tcat, W1, b1c, W2, b2c, W3, b3c)

    out_sc = pl.pallas_call(
        _mlp_body,
        grid=(1,),
        in_specs=[full((128, _BSC)),
                  full((128, 64)), full((64, 1)), full((64, 32)),
                  full((32, 1)), full((32, 1)), full((1, 1))],
        out_specs=full((1, _BSC)),
        out_shape=jax.ShapeDtypeStruct((1, _BSC), f32),
    )(comb_sc, W1, b1c, W2, b2c, W3, b3c)

    return jnp.concatenate(
        [out_sc.reshape(_BSC, 1), out_tc.reshape(_BTC, 1)], axis=0)
